# Initial kernel scaffold; baseline (speedup 1.0000x reference)
import jax
import jax.numpy as jnp
from jax.experimental import pallas as pl


def kernel(node_embeddings, missing_mask, edge_index, W1, b1, W2, b2):
    raise NotImplementedError("write your pallas kernel here")


# speedup vs baseline: 48.1524x; 48.1524x over previous
import functools

import jax
import jax.numpy as jnp
from jax import lax
from jax.experimental import pallas as pl
from jax.experimental.pallas import tpu as pltpu
from jax.experimental.pallas import tpu_sc as plsc

B = 4
N = 10000
H = 128
E = 320000

NC = 2
NS = 16

EDGES_PER_TILE = E // NS
CHUNK = 125
NCHUNK = EDGES_PER_TILE // CHUNK
IDXBLK = 32
ROWS_MAIN = 624
ROWS_LAST = N - (NS - 1) * ROWS_MAIN

_sc_mesh = plsc.VectorSubcoreMesh(core_axis_name="c", subcore_axis_name="s")


@functools.partial(
    pl.kernel,
    out_type=jax.ShapeDtypeStruct((B * N, H), jnp.float32),
    mesh=_sc_mesh,
    scratch_types=[
        pltpu.VMEM((IDXBLK, CHUNK), jnp.int32),
        pltpu.VMEM((IDXBLK, CHUNK), jnp.int32),
        pltpu.VMEM((CHUNK, H), jnp.float32),
        pltpu.VMEM_SHARED((N, H), jnp.float32),
        pltpu.SemaphoreType.DMA,
    ],
)
def _sc_segment_sum(emb, src_b, dst_t, zeros, out, src_v, dst_v, gbuf, acc, sem):
    c = lax.axis_index("c")
    s = lax.axis_index("s")
    row0 = s * ROWS_MAIN
    for k in range(B // NC):
        b = NC * c + k

        @pl.when(s < NS - 1)
        def _():
            pltpu.sync_copy(zeros.at[pl.ds(0, ROWS_MAIN)],
                            acc.at[pl.ds(row0, ROWS_MAIN)])

        @pl.when(s == NS - 1)
        def _():
            pltpu.sync_copy(zeros, acc.at[pl.ds((NS - 1) * ROWS_MAIN, ROWS_LAST)])

        plsc.subcore_barrier()

        def blk_body(i, carry):
            pltpu.sync_copy(src_b.at[b].at[s].at[pl.ds(i * IDXBLK, IDXBLK)], src_v)
            pltpu.sync_copy(dst_t.at[s].at[pl.ds(i * IDXBLK, IDXBLK)], dst_v)

            def chunk_body(j, carry2):
                pltpu.async_copy(emb.at[src_v.at[j]], gbuf, sem).wait()
                pltpu.sync_copy(gbuf, acc.at[dst_v.at[j]], add=True)
                return carry2

            lax.fori_loop(0, IDXBLK, chunk_body, 0)
            return carry

        lax.fori_loop(0, NCHUNK // IDXBLK, blk_body, 0)
        plsc.subcore_barrier()

        @pl.when(s < NS - 1)
        def _():
            pltpu.sync_copy(acc.at[pl.ds(row0, ROWS_MAIN)],
                            out.at[pl.ds(b * N + row0, ROWS_MAIN)])

        @pl.when(s == NS - 1)
        def _():
            pltpu.sync_copy(
                acc.at[pl.ds((NS - 1) * ROWS_MAIN, ROWS_LAST)],
                out.at[pl.ds(b * N + (NS - 1) * ROWS_MAIN, ROWS_LAST)])

        plsc.subcore_barrier()


def _mlp_body(nb_ref, x_ref, m_ref, w1a_ref, w1b_ref, b1_ref, w2_ref, b2_ref, out_ref):
    h = jnp.dot(nb_ref[...], w1a_ref[...], preferred_element_type=jnp.float32)
    h += jnp.dot(x_ref[...], w1b_ref[...], preferred_element_type=jnp.float32)
    h = jnp.maximum(h + b1_ref[...], 0.0)
    imp = jnp.dot(h, w2_ref[...], preferred_element_type=jnp.float32) + b2_ref[...]
    out_ref[...] = jnp.where(m_ref[...] != 0, imp, x_ref[...])


MLP_BLK = 2000


def _mlp(nb, x, m, w1a, w1b, b1, w2, b2):
    grid = ((B * N) // MLP_BLK,)
    return pl.pallas_call(
        _mlp_body,
        grid=grid,
        in_specs=[
            pl.BlockSpec((MLP_BLK, H), lambda i: (i, 0)),
            pl.BlockSpec((MLP_BLK, H), lambda i: (i, 0)),
            pl.BlockSpec((MLP_BLK, 1), lambda i: (i, 0)),
            pl.BlockSpec((H, H), lambda i: (0, 0)),
            pl.BlockSpec((H, H), lambda i: (0, 0)),
            pl.BlockSpec((1, H), lambda i: (0, 0)),
            pl.BlockSpec((H, H), lambda i: (0, 0)),
            pl.BlockSpec((1, H), lambda i: (0, 0)),
        ],
        out_specs=pl.BlockSpec((MLP_BLK, H), lambda i: (i, 0)),
        out_shape=jax.ShapeDtypeStruct((B * N, H), jnp.float32),
    )(nb, x, m, w1a, w1b, b1, w2, b2)


@jax.jit
def kernel(node_embeddings, missing_mask, edge_index, W1, b1, W2, b2):
    src = edge_index[0].astype(jnp.int32)
    dst = edge_index[1].astype(jnp.int32)
    emb_flat = node_embeddings.reshape(B * N, H)
    offs = (jnp.arange(B, dtype=jnp.int32) * N)[:, None]
    src_b = (src[None, :] + offs).reshape(B, NS, NCHUNK, CHUNK)
    dst_t = dst.reshape(NS, NCHUNK, CHUNK)
    zeros = jnp.zeros((ROWS_LAST, H), jnp.float32)
    nb_flat = _sc_segment_sum(emb_flat, src_b, dst_t, zeros)
    mask = missing_mask.reshape(B * N, 1).astype(jnp.int32)
    out_flat = _mlp(nb_flat, emb_flat, mask, W1[:H], W1[H:], b1.reshape(1, H),
                    W2, b2.reshape(1, H))
    return out_flat.reshape(B, N, H)


# speedup vs baseline: 61.3807x; 1.2747x over previous
import functools

import jax
import jax.numpy as jnp
from jax import lax
from jax.experimental import pallas as pl
from jax.experimental.pallas import tpu as pltpu
from jax.experimental.pallas import tpu_sc as plsc

B = 4
N = 10000
H = 128
E = 320000

NC = 2
NS = 16

EDGES_PER_TILE = E // NS
CHUNK = 125
NCHUNK = EDGES_PER_TILE // CHUNK
IDXBLK = 16
ROWS_MAIN = 624
ROWS_LAST = N - (NS - 1) * ROWS_MAIN

_sc_mesh = plsc.VectorSubcoreMesh(core_axis_name="c", subcore_axis_name="s")


@functools.partial(
    pl.kernel,
    out_type=jax.ShapeDtypeStruct((B * N, H), jnp.float32),
    mesh=_sc_mesh,
    scratch_types=[
        pltpu.VMEM((IDXBLK, CHUNK), jnp.int32),
        pltpu.VMEM((IDXBLK, CHUNK), jnp.int32),
        pltpu.VMEM((CHUNK, H), jnp.float32),
        pltpu.VMEM((CHUNK, H), jnp.float32),
        pltpu.VMEM_SHARED((N, H), jnp.float32),
        pltpu.SemaphoreType.DMA,
        pltpu.SemaphoreType.DMA,
    ],
)
def _sc_segment_sum(emb, src_b, dst_t, zeros, out,
                    src_v, dst_v, gbuf0, gbuf1, acc, sem0, sem1):
    c = lax.axis_index("c")
    s = lax.axis_index("s")
    row0 = s * ROWS_MAIN
    for k in range(B // NC):
        b = NC * c + k

        @pl.when(s < NS - 1)
        def _():
            pltpu.sync_copy(zeros.at[pl.ds(0, ROWS_MAIN)],
                            acc.at[pl.ds(row0, ROWS_MAIN)])

        @pl.when(s == NS - 1)
        def _():
            pltpu.sync_copy(zeros, acc.at[pl.ds((NS - 1) * ROWS_MAIN, ROWS_LAST)])

        plsc.subcore_barrier()

        bufs = (gbuf0, gbuf1)
        sems = (sem0, sem1)

        def blk_body(i, carry):
            pltpu.sync_copy(src_b.at[b].at[s].at[pl.ds(i * IDXBLK, IDXBLK)], src_v)
            pltpu.sync_copy(dst_t.at[s].at[pl.ds(i * IDXBLK, IDXBLK)], dst_v)
            cps = [pltpu.async_copy(emb.at[src_v.at[0]], bufs[0], sems[0])]
            for j in range(IDXBLK):
                cps[j].wait()
                if j + 1 < IDXBLK:
                    cps.append(pltpu.async_copy(
                        emb.at[src_v.at[j + 1]], bufs[(j + 1) % 2], sems[(j + 1) % 2]))
                pltpu.sync_copy(bufs[j % 2], acc.at[dst_v.at[j]], add=True)
            return carry

        lax.fori_loop(0, NCHUNK // IDXBLK, blk_body, 0)
        plsc.subcore_barrier()

        @pl.when(s < NS - 1)
        def _():
            pltpu.sync_copy(acc.at[pl.ds(row0, ROWS_MAIN)],
                            out.at[pl.ds(b * N + row0, ROWS_MAIN)])

        @pl.when(s == NS - 1)
        def _():
            pltpu.sync_copy(
                acc.at[pl.ds((NS - 1) * ROWS_MAIN, ROWS_LAST)],
                out.at[pl.ds(b * N + (NS - 1) * ROWS_MAIN, ROWS_LAST)])

        plsc.subcore_barrier()


def _mlp_body(nb_ref, x_ref, m_ref, w1a_ref, w1b_ref, b1_ref, w2_ref, b2_ref, out_ref):
    h = jnp.dot(nb_ref[...], w1a_ref[...], preferred_element_type=jnp.float32)
    h += jnp.dot(x_ref[...], w1b_ref[...], preferred_element_type=jnp.float32)
    h = jnp.maximum(h + b1_ref[...], 0.0)
    imp = jnp.dot(h, w2_ref[...], preferred_element_type=jnp.float32) + b2_ref[...]
    out_ref[...] = jnp.where(m_ref[...] != 0, imp, x_ref[...])


MLP_BLK = 2000


def _mlp(nb, x, m, w1a, w1b, b1, w2, b2):
    grid = ((B * N) // MLP_BLK,)
    return pl.pallas_call(
        _mlp_body,
        grid=grid,
        in_specs=[
            pl.BlockSpec((MLP_BLK, H), lambda i: (i, 0)),
            pl.BlockSpec((MLP_BLK, H), lambda i: (i, 0)),
            pl.BlockSpec((MLP_BLK, 1), lambda i: (i, 0)),
            pl.BlockSpec((H, H), lambda i: (0, 0)),
            pl.BlockSpec((H, H), lambda i: (0, 0)),
            pl.BlockSpec((1, H), lambda i: (0, 0)),
            pl.BlockSpec((H, H), lambda i: (0, 0)),
            pl.BlockSpec((1, H), lambda i: (0, 0)),
        ],
        out_specs=pl.BlockSpec((MLP_BLK, H), lambda i: (i, 0)),
        out_shape=jax.ShapeDtypeStruct((B * N, H), jnp.float32),
    )(nb, x, m, w1a, w1b, b1, w2, b2)


@jax.jit
def kernel(node_embeddings, missing_mask, edge_index, W1, b1, W2, b2):
    src = edge_index[0].astype(jnp.int32)
    dst = edge_index[1].astype(jnp.int32)
    emb_flat = node_embeddings.reshape(B * N, H)
    offs = (jnp.arange(B, dtype=jnp.int32) * N)[:, None]
    src_b = (src[None, :] + offs).reshape(B, NS, NCHUNK, CHUNK)
    dst_t = dst.reshape(NS, NCHUNK, CHUNK)
    zeros = jnp.zeros((ROWS_LAST, H), jnp.float32)
    nb_flat = _sc_segment_sum(emb_flat, src_b, dst_t, zeros)
    mask = missing_mask.reshape(B * N, 1).astype(jnp.int32)
    out_flat = _mlp(nb_flat, emb_flat, mask, W1[:H], W1[H:], b1.reshape(1, H),
                    W2, b2.reshape(1, H))
    return out_flat.reshape(B, N, H)
